# baseline (speedup 1.0000x reference)
import functools

import jax
import jax.numpy as jnp
from jax import lax
from jax.experimental import pallas as pl
from jax.experimental.pallas import tpu as pltpu
from jax.experimental.pallas import tpu_sc as plsc

D = 64
DP = 128
P = 20
L = 16
NC = 2
NS = 16
NW = NC * NS
WBLK = 12800


def _widen_body(wt_ref, out_ref):
    blk = wt_ref[...]
    t = jnp.transpose(blk)
    out_ref[...] = jnp.concatenate([t, t], axis=1)


def _widen(W):
    V = W.shape[0]
    return pl.pallas_call(
        _widen_body,
        grid=((V + WBLK - 1) // WBLK,),
        in_specs=[pl.BlockSpec((D, WBLK), lambda i: (0, i))],
        out_specs=pl.BlockSpec((WBLK, DP), lambda i: (i, 0)),
        out_shape=jax.ShapeDtypeStruct((V, DP), jnp.float32),
    )(W.T)


def _sc_losses(pu, pv, nv, W2):
    B = pu.shape[0]
    BPW = B // NW
    NCH = BPW // L
    E = L * P
    inv_b = 1.0 / B

    mesh = plsc.VectorSubcoreMesh(
        core_axis_name="c", subcore_axis_name="s", num_cores=NC, num_subcores=NS
    )

    GPC = E // 64
    NR = BPW * P // 64

    @functools.partial(
        pl.kernel,
        mesh=mesh,
        compiler_params=pltpu.CompilerParams(
            needs_layout_passes=False,
            use_tc_tiling_on_sc=False,
            disable_bounds_checks=True,
        ),
        out_type=[
            jax.ShapeDtypeStruct((B,), jnp.float32),
            jax.ShapeDtypeStruct((B,), jnp.float32),
        ],
        scratch_types=[
            pltpu.VMEM((BPW,), jnp.int32),
            pltpu.VMEM((NR, 64), jnp.int32),
            pltpu.VMEM((NR, 64), jnp.int32),
            pltpu.VMEM((BPW, D), jnp.float32),
            pltpu.VMEM((E, D), jnp.float32),
            pltpu.VMEM((E, D), jnp.float32),
            pltpu.VMEM((BPW,), jnp.float32),
            pltpu.VMEM((BPW,), jnp.float32),
            pltpu.SemaphoreType.DMA,
            pltpu.SemaphoreType.DMA,
            pltpu.SemaphoreType.DMA,
        ],
    )
    def body(pu_hbm, pv_hbm, nv_hbm, w_hbm, pos_out, neg_out,
             idx_u, idx_p, idx_n, rows_u, rows_p, rows_n,
             out_pos_v, out_neg_v, sem_u, sem_p, sem_n):
        wid = lax.axis_index("s") * NC + lax.axis_index("c")
        base = wid * BPW

        pltpu.sync_copy(pu_hbm.at[pl.ds(base, BPW)], idx_u)
        pltpu.sync_copy(pv_hbm.at[pl.ds(wid * NR, NR)], idx_p)
        pltpu.sync_copy(nv_hbm.at[pl.ds(wid * NR, NR)], idx_n)
        cp_u = pltpu.async_copy(w_hbm.at[idx_u], rows_u, sem_u)

        def fire(idx2d, rows, sem, c):
            return [
                pltpu.async_copy(
                    w_hbm.at[idx2d.at[GPC * c + j]],
                    rows.at[pl.ds(64 * j, 64)],
                    sem,
                )
                for j in range(GPC)
            ]

        iota = lax.iota(jnp.int32, L)
        prow = iota * P
        inflight_p = fire(idx_p, rows_p, sem_p, 0)
        inflight_n = fire(idx_n, rows_n, sem_n, 0)
        cp_u.wait()

        zeros = jnp.zeros((L,), jnp.float32)
        for c in range(NCH):
            urow = c * L + iota

            for cp in inflight_p:
                cp.wait()

            @plsc.parallel_loop(0, D, 1, unroll=4, carry=zeros)
            def acc_pos(d, acc):
                col = jnp.full((L,), d, dtype=jnp.int32)
                u = plsc.load_gather(rows_u, [urow, col])
                g = [
                    plsc.load_gather(rows_p, [prow + p, col])
                    for p in range(P)
                ]
                while len(g) > 1:
                    g = [
                        g[i] + g[i + 1] if i + 1 < len(g) else g[i]
                        for i in range(0, len(g), 2)
                    ]
                return acc + u * g[0]
            out_pos_v[pl.ds(c * L, L)] = acc_pos * inv_b
            if c + 1 < NCH:
                inflight_p = fire(idx_p, rows_p, sem_p, c + 1)

            for cp in inflight_n:
                cp.wait()

            @plsc.parallel_loop(0, D, 1, unroll=4, carry=(zeros,) * P)
            def res(d, carry):
                col = jnp.full((L,), d, dtype=jnp.int32)
                u = plsc.load_gather(rows_u, [urow, col])
                return tuple(
                    carry[n] + u * plsc.load_gather(rows_n, [prow + n, col])
                    for n in range(P)
                )
            neg_vec = zeros
            for n in range(P):
                s = res[n] * inv_b
                neg_vec = neg_vec + jnp.where(s > 0.0, jnp.exp(s), 0.0)
            out_neg_v[pl.ds(c * L, L)] = neg_vec
            if c + 1 < NCH:
                inflight_n = fire(idx_n, rows_n, sem_n, c + 1)

        pltpu.sync_copy(out_pos_v, pos_out.at[pl.ds(base, BPW)])
        pltpu.sync_copy(out_neg_v, neg_out.at[pl.ds(base, BPW)])

    return body(pu, pv, nv, W2)


def _combine_body(pos_ref, neg_ref, out_ref):
    out_ref[...] = jnp.log(1.0 + neg_ref[...]) - pos_ref[...]


def kernel(pos_u, pos_v, neg_v, W):
    B = pos_u.shape[0]
    W2 = _widen(W)
    W3 = W2.reshape(2 * W.shape[0], D)
    pos_loss, neg_raw = _sc_losses(
        (pos_u.reshape(-1) * 2).astype(jnp.int32),
        (pos_v.reshape(-1, 64) * 2).astype(jnp.int32),
        (neg_v.reshape(-1, 64) * 2).astype(jnp.int32),
        W3,
    )
    out = pl.pallas_call(
        _combine_body,
        out_shape=jax.ShapeDtypeStruct((B // 128, 128), jnp.float32),
    )(pos_loss.reshape(B // 128, 128), neg_raw.reshape(B // 128, 128))
    return out.reshape(B)


# speedup vs baseline: 1.0475x; 1.0475x over previous
import functools

import jax
import jax.numpy as jnp
from jax import lax
from jax.experimental import pallas as pl
from jax.experimental.pallas import tpu as pltpu
from jax.experimental.pallas import tpu_sc as plsc

D = 64
DP = 128
P = 20
L = 16
NC = 2
NS = 16
NW = NC * NS
WBLK = 12800


def _widen_body(wt_ref, out_ref):
    blk = wt_ref[...]
    t = jnp.transpose(blk)
    out_ref[...] = jnp.concatenate([t, t], axis=1)


def _widen(W):
    V = W.shape[0]
    return pl.pallas_call(
        _widen_body,
        grid=((V + WBLK - 1) // WBLK,),
        in_specs=[pl.BlockSpec((D, WBLK), lambda i: (0, i))],
        out_specs=pl.BlockSpec((WBLK, DP), lambda i: (i, 0)),
        out_shape=jax.ShapeDtypeStruct((V, DP), jnp.float32),
    )(W.T)


def _sc_losses(pu, pv, nv, W2):
    B = pu.shape[0]
    BPW = B // NW
    NCH = BPW // L
    E = L * P
    inv_b = 1.0 / B

    mesh = plsc.VectorSubcoreMesh(
        core_axis_name="c", subcore_axis_name="s", num_cores=NC, num_subcores=NS
    )

    GPC = E // 64
    NR = BPW * P // 64

    @functools.partial(
        pl.kernel,
        mesh=mesh,
        compiler_params=pltpu.CompilerParams(
            needs_layout_passes=False,
            use_tc_tiling_on_sc=False,
            disable_bounds_checks=True,
        ),
        out_type=[
            jax.ShapeDtypeStruct((B,), jnp.float32),
            jax.ShapeDtypeStruct((B,), jnp.float32),
        ],
        scratch_types=[
            pltpu.VMEM((BPW,), jnp.int32),
            pltpu.VMEM((NR, 64), jnp.int32),
            pltpu.VMEM((NR, 64), jnp.int32),
            pltpu.VMEM((BPW, D), jnp.float32),
            pltpu.VMEM((E, D), jnp.float32),
            pltpu.VMEM((E, D), jnp.float32),
            pltpu.VMEM((BPW,), jnp.float32),
            pltpu.VMEM((BPW,), jnp.float32),
            pltpu.SemaphoreType.DMA,
            pltpu.SemaphoreType.DMA,
            pltpu.SemaphoreType.DMA,
        ],
    )
    def body(pu_hbm, pv_hbm, nv_hbm, w_hbm, pos_out, neg_out,
             idx_u, idx_p, idx_n, rows_u, rows_p, rows_n,
             out_pos_v, out_neg_v, sem_u, sem_p, sem_n):
        wid = lax.axis_index("s") * NC + lax.axis_index("c")
        base = wid * BPW

        pltpu.sync_copy(pu_hbm.at[pl.ds(base, BPW)], idx_u)
        pltpu.sync_copy(pv_hbm.at[pl.ds(wid * NR, NR)], idx_p)
        pltpu.sync_copy(nv_hbm.at[pl.ds(wid * NR, NR)], idx_n)
        cp_u = pltpu.async_copy(w_hbm.at[idx_u], rows_u, sem_u)

        def fire(idx2d, rows, sem, c):
            return [
                pltpu.async_copy(
                    w_hbm.at[idx2d.at[GPC * c + j]],
                    rows.at[pl.ds(64 * j, 64)],
                    sem,
                )
                for j in range(GPC)
            ]

        iota = lax.iota(jnp.int32, L)
        prow = iota * P
        inflight_p = fire(idx_p, rows_p, sem_p, 0)
        inflight_n = fire(idx_n, rows_n, sem_n, 0)
        cp_u.wait()

        zeros = jnp.zeros((L,), jnp.float32)
        for c in range(NCH):
            urow = c * L + iota

            for cp in inflight_p:
                cp.wait()

            def pbody(d, acc):
                col = jnp.full((L,), d, dtype=jnp.int32)
                u = plsc.load_gather(rows_u, [urow, col])
                ps = plsc.load_gather(rows_p, [prow, col])
                for p in range(1, P):
                    ps = ps + plsc.load_gather(rows_p, [prow + p, col])
                return acc + u * ps

            acc_pos = lax.fori_loop(0, D, pbody, zeros)
            out_pos_v[pl.ds(c * L, L)] = acc_pos * inv_b
            if c + 1 < NCH:
                inflight_p = fire(idx_p, rows_p, sem_p, c + 1)

            for cp in inflight_n:
                cp.wait()

            def nbody(d, carry):
                col = jnp.full((L,), d, dtype=jnp.int32)
                u = plsc.load_gather(rows_u, [urow, col])
                return tuple(
                    carry[n] + u * plsc.load_gather(rows_n, [prow + n, col])
                    for n in range(P)
                )

            res = lax.fori_loop(0, D, nbody, (zeros,) * P)
            neg_vec = zeros
            for n in range(P):
                s = res[n] * inv_b
                neg_vec = neg_vec + jnp.where(s > 0.0, jnp.exp(s), 0.0)
            out_neg_v[pl.ds(c * L, L)] = neg_vec
            if c + 1 < NCH:
                inflight_n = fire(idx_n, rows_n, sem_n, c + 1)

        pltpu.sync_copy(out_pos_v, pos_out.at[pl.ds(base, BPW)])
        pltpu.sync_copy(out_neg_v, neg_out.at[pl.ds(base, BPW)])

    return body(pu, pv, nv, W2)


def _combine_body(pos_ref, neg_ref, out_ref):
    out_ref[...] = jnp.log(1.0 + neg_ref[...]) - pos_ref[...]


def kernel(pos_u, pos_v, neg_v, W):
    B = pos_u.shape[0]
    W2 = _widen(W)
    W3 = W2.reshape(2 * W.shape[0], D)
    pos_loss, neg_raw = _sc_losses(
        (pos_u.reshape(-1) * 2).astype(jnp.int32),
        (pos_v.reshape(-1, 64) * 2).astype(jnp.int32),
        (neg_v.reshape(-1, 64) * 2).astype(jnp.int32),
        W3,
    )
    out = pl.pallas_call(
        _combine_body,
        out_shape=jax.ShapeDtypeStruct((B // 128, 128), jnp.float32),
    )(pos_loss.reshape(B // 128, 128), neg_raw.reshape(B // 128, 128))
    return out.reshape(B)


# speedup vs baseline: 1.1582x; 1.1057x over previous
import functools

import jax
import jax.numpy as jnp
from jax import lax
from jax.experimental import pallas as pl
from jax.experimental.pallas import tpu as pltpu
from jax.experimental.pallas import tpu_sc as plsc

D = 64
DP = 128
P = 20
L = 16
NC = 2
NS = 16
NW = NC * NS
WBLK = 12800


def _widen_body(wt_ref, out_ref):
    blk = wt_ref[...]
    out_ref[:, 0:D] = jnp.transpose(blk)


def _widen(W):
    V = W.shape[0]
    return pl.pallas_call(
        _widen_body,
        grid=((V + WBLK - 1) // WBLK,),
        in_specs=[pl.BlockSpec((D, WBLK), lambda i: (0, i))],
        out_specs=pl.BlockSpec((WBLK, DP), lambda i: (i, 0)),
        out_shape=jax.ShapeDtypeStruct((V, DP), jnp.float32),
    )(W.T)


def _sc_losses(pu, pv, nv, W2):
    B = pu.shape[0]
    BPW = B // NW
    NCH = BPW // L
    E = L * P
    inv_b = 1.0 / B

    mesh = plsc.VectorSubcoreMesh(
        core_axis_name="c", subcore_axis_name="s", num_cores=NC, num_subcores=NS
    )

    GPC = E // 64
    NR = BPW * P // 64

    @functools.partial(
        pl.kernel,
        mesh=mesh,
        compiler_params=pltpu.CompilerParams(
            needs_layout_passes=False,
            use_tc_tiling_on_sc=False,
            disable_bounds_checks=True,
        ),
        out_type=[
            jax.ShapeDtypeStruct((B,), jnp.float32),
            jax.ShapeDtypeStruct((B,), jnp.float32),
        ],
        scratch_types=[
            pltpu.VMEM((BPW,), jnp.int32),
            pltpu.VMEM((NR, 64), jnp.int32),
            pltpu.VMEM((NR, 64), jnp.int32),
            pltpu.VMEM((BPW, D), jnp.float32),
            pltpu.VMEM((E, D), jnp.float32),
            pltpu.VMEM((E, D), jnp.float32),
            pltpu.VMEM((BPW,), jnp.float32),
            pltpu.VMEM((BPW,), jnp.float32),
            pltpu.SemaphoreType.DMA,
            pltpu.SemaphoreType.DMA,
            pltpu.SemaphoreType.DMA,
        ],
    )
    def body(pu_hbm, pv_hbm, nv_hbm, w_hbm, pos_out, neg_out,
             idx_u, idx_p, idx_n, rows_u, rows_p, rows_n,
             out_pos_v, out_neg_v, sem_u, sem_p, sem_n):
        wid = lax.axis_index("s") * NC + lax.axis_index("c")
        base = wid * BPW

        pltpu.sync_copy(pu_hbm.at[pl.ds(base, BPW)], idx_u)
        pltpu.sync_copy(pv_hbm.at[pl.ds(wid * NR, NR)], idx_p)
        pltpu.sync_copy(nv_hbm.at[pl.ds(wid * NR, NR)], idx_n)
        cp_u = pltpu.async_copy(w_hbm.at[idx_u], rows_u, sem_u)

        def fire(idx2d, rows, sem, c):
            return [
                pltpu.async_copy(
                    w_hbm.at[idx2d.at[GPC * c + j]],
                    rows.at[pl.ds(64 * j, 64)],
                    sem,
                )
                for j in range(GPC)
            ]

        iota = lax.iota(jnp.int32, L)
        prow = iota * P
        inflight_p = fire(idx_p, rows_p, sem_p, 0)
        inflight_n = fire(idx_n, rows_n, sem_n, 0)
        cp_u.wait()

        zeros = jnp.zeros((L,), jnp.float32)
        for c in range(NCH):
            urow = c * L + iota

            for cp in inflight_p:
                cp.wait()

            def pbody(d, acc):
                col = jnp.full((L,), d, dtype=jnp.int32)
                u = plsc.load_gather(rows_u, [urow, col])
                ps = plsc.load_gather(rows_p, [prow, col])
                for p in range(1, P):
                    ps = ps + plsc.load_gather(rows_p, [prow + p, col])
                return acc + u * ps

            acc_pos = lax.fori_loop(0, D, pbody, zeros)
            out_pos_v[pl.ds(c * L, L)] = acc_pos * inv_b
            if c + 1 < NCH:
                inflight_p = fire(idx_p, rows_p, sem_p, c + 1)

            for cp in inflight_n:
                cp.wait()

            def nbody(d, carry):
                col = jnp.full((L,), d, dtype=jnp.int32)
                u = plsc.load_gather(rows_u, [urow, col])
                return tuple(
                    carry[n] + u * plsc.load_gather(rows_n, [prow + n, col])
                    for n in range(P)
                )

            res = lax.fori_loop(0, D, nbody, (zeros,) * P)
            neg_vec = zeros
            for n in range(P):
                s = res[n] * inv_b
                neg_vec = neg_vec + jnp.where(s > 0.0, jnp.exp(s), 0.0)
            out_neg_v[pl.ds(c * L, L)] = neg_vec
            if c + 1 < NCH:
                inflight_n = fire(idx_n, rows_n, sem_n, c + 1)

        pltpu.sync_copy(out_pos_v, pos_out.at[pl.ds(base, BPW)])
        pltpu.sync_copy(out_neg_v, neg_out.at[pl.ds(base, BPW)])

    return body(pu, pv, nv, W2)


def _combine_body(pos_ref, neg_ref, out_ref):
    out_ref[...] = jnp.log(1.0 + neg_ref[...]) - pos_ref[...]


def kernel(pos_u, pos_v, neg_v, W):
    B = pos_u.shape[0]
    W2 = _widen(W)
    W3 = W2.reshape(2 * W.shape[0], D)
    pos_loss, neg_raw = _sc_losses(
        (pos_u.reshape(-1) * 2).astype(jnp.int32),
        (pos_v.reshape(-1, 64) * 2).astype(jnp.int32),
        (neg_v.reshape(-1, 64) * 2).astype(jnp.int32),
        W3,
    )
    out = pl.pallas_call(
        _combine_body,
        out_shape=jax.ShapeDtypeStruct((B // 128, 128), jnp.float32),
    )(pos_loss.reshape(B // 128, 128), neg_raw.reshape(B // 128, 128))
    return out.reshape(B)


# speedup vs baseline: 1.1781x; 1.0172x over previous
import functools

import jax
import jax.numpy as jnp
from jax import lax
from jax.experimental import pallas as pl
from jax.experimental.pallas import tpu as pltpu
from jax.experimental.pallas import tpu_sc as plsc

D = 64
DP = 128
P = 20
L = 16
NC = 2
NS = 16
NW = NC * NS
WBLK = 25600


def _widen_body(wt_ref, out_ref):
    blk = wt_ref[...]
    out_ref[:, 0:D] = jnp.transpose(blk)


def _widen(W):
    V = W.shape[0]
    return pl.pallas_call(
        _widen_body,
        grid=((V + WBLK - 1) // WBLK,),
        in_specs=[pl.BlockSpec((D, WBLK), lambda i: (0, i))],
        out_specs=pl.BlockSpec((WBLK, DP), lambda i: (i, 0)),
        out_shape=jax.ShapeDtypeStruct((V, DP), jnp.float32),
    )(W.T)


def _sc_losses(pu, pv, nv, W2):
    B = pu.shape[0]
    BPW = B // NW
    NCH = BPW // L
    E = L * P
    inv_b = 1.0 / B

    mesh = plsc.VectorSubcoreMesh(
        core_axis_name="c", subcore_axis_name="s", num_cores=NC, num_subcores=NS
    )

    GPC = E // 64
    NR = BPW * P // 64

    @functools.partial(
        pl.kernel,
        mesh=mesh,
        compiler_params=pltpu.CompilerParams(
            needs_layout_passes=False,
            use_tc_tiling_on_sc=False,
            disable_bounds_checks=True,
        ),
        out_type=[
            jax.ShapeDtypeStruct((B,), jnp.float32),
            jax.ShapeDtypeStruct((B,), jnp.float32),
        ],
        scratch_types=[
            pltpu.VMEM((BPW,), jnp.int32),
            pltpu.VMEM((NR, 64), jnp.int32),
            pltpu.VMEM((NR, 64), jnp.int32),
            pltpu.VMEM((BPW, D), jnp.float32),
            pltpu.VMEM((E, D), jnp.float32),
            pltpu.VMEM((E, D), jnp.float32),
            pltpu.VMEM((BPW,), jnp.float32),
            pltpu.VMEM((BPW,), jnp.float32),
            pltpu.SemaphoreType.DMA,
            pltpu.SemaphoreType.DMA,
            pltpu.SemaphoreType.DMA,
        ],
    )
    def body(pu_hbm, pv_hbm, nv_hbm, w_hbm, pos_out, neg_out,
             idx_u, idx_p, idx_n, rows_u, rows_p, rows_n,
             out_pos_v, out_neg_v, sem_u, sem_p, sem_n):
        wid = lax.axis_index("s") * NC + lax.axis_index("c")
        base = wid * BPW

        pltpu.sync_copy(pu_hbm.at[pl.ds(base, BPW)], idx_u)
        pltpu.sync_copy(pv_hbm.at[pl.ds(wid * NR, NR)], idx_p)
        pltpu.sync_copy(nv_hbm.at[pl.ds(wid * NR, NR)], idx_n)
        cp_u = pltpu.async_copy(w_hbm.at[idx_u], rows_u, sem_u)

        def fire(idx2d, rows, sem, c):
            return [
                pltpu.async_copy(
                    w_hbm.at[idx2d.at[GPC * c + j]],
                    rows.at[pl.ds(64 * j, 64)],
                    sem,
                )
                for j in range(GPC)
            ]

        iota = lax.iota(jnp.int32, L)
        prow = iota * P
        inflight_p = fire(idx_p, rows_p, sem_p, 0)
        inflight_n = fire(idx_n, rows_n, sem_n, 0)
        cp_u.wait()

        zeros = jnp.zeros((L,), jnp.float32)
        for c in range(NCH):
            urow = c * L + iota

            for cp in inflight_p:
                cp.wait()

            def pbody(d, acc):
                col = jnp.full((L,), d, dtype=jnp.int32)
                u = plsc.load_gather(rows_u, [urow, col])
                ps = plsc.load_gather(rows_p, [prow, col])
                for p in range(1, P):
                    ps = ps + plsc.load_gather(rows_p, [prow + p, col])
                return acc + u * ps

            acc_pos = lax.fori_loop(0, D, pbody, zeros)
            out_pos_v[pl.ds(c * L, L)] = acc_pos * inv_b
            if c + 1 < NCH:
                inflight_p = fire(idx_p, rows_p, sem_p, c + 1)

            for cp in inflight_n:
                cp.wait()

            def nbody(d, carry):
                col = jnp.full((L,), d, dtype=jnp.int32)
                u = plsc.load_gather(rows_u, [urow, col])
                return tuple(
                    carry[n] + u * plsc.load_gather(rows_n, [prow + n, col])
                    for n in range(P)
                )

            res = lax.fori_loop(0, D, nbody, (zeros,) * P)
            neg_vec = zeros
            for n in range(P):
                s = res[n] * inv_b
                neg_vec = neg_vec + jnp.where(s > 0.0, jnp.exp(s), 0.0)
            out_neg_v[pl.ds(c * L, L)] = neg_vec
            if c + 1 < NCH:
                inflight_n = fire(idx_n, rows_n, sem_n, c + 1)

        pltpu.sync_copy(out_pos_v, pos_out.at[pl.ds(base, BPW)])
        pltpu.sync_copy(out_neg_v, neg_out.at[pl.ds(base, BPW)])

    return body(pu, pv, nv, W2)


def _combine_body(pos_ref, neg_ref, out_ref):
    out_ref[...] = jnp.log(1.0 + neg_ref[...]) - pos_ref[...]


def kernel(pos_u, pos_v, neg_v, W):
    B = pos_u.shape[0]
    W2 = _widen(W)
    W3 = W2.reshape(2 * W.shape[0], D)
    pos_loss, neg_raw = _sc_losses(
        (pos_u.reshape(-1) * 2).astype(jnp.int32),
        (pos_v.reshape(-1, 64) * 2).astype(jnp.int32),
        (neg_v.reshape(-1, 64) * 2).astype(jnp.int32),
        W3,
    )
    out = pl.pallas_call(
        _combine_body,
        out_shape=jax.ShapeDtypeStruct((B // 128, 128), jnp.float32),
    )(pos_loss.reshape(B // 128, 128), neg_raw.reshape(B // 128, 128))
    return out.reshape(B)
